# baseline (speedup 1.0000x reference)
import functools

import jax
import jax.numpy as jnp
from jax import lax
from jax.experimental import pallas as pl
from jax.experimental.pallas import tpu as pltpu

_B = 8
_N = 128
_ND = 2 * _N - 1
_BIG = 10000000000.0
_LOG2E = 1.4426950408889634
_A = 4


def _rot_r(x, s):
    return jnp.concatenate([x[:, -s:], x[:, :-s]], axis=1)


def _rot_l(x, s):
    return jnp.concatenate([x[:, s:], x[:, :s]], axis=1)


def _dtw_kernel(dt_ref, out_ref, *scratch):
    f32 = jnp.float32
    tsk_refs = scratch[:_A]
    qas_refs = scratch[_A : 2 * _A]
    qbs_refs = scratch[2 * _A : 3 * _A]
    qc_refs = scratch[3 * _A : 4 * _A]
    esk_ref = scratch[4 * _A]

    cur = dt_ref[...] * f32(_LOG2E)
    lane3 = lax.broadcasted_iota(jnp.int32, (_N, _B, _N), 2)
    for t in range(7):
        s = 1 << t
        rolled = jnp.concatenate([cur[_N - s :], cur[: _N - s]], axis=0)
        cur = jnp.where((lane3 >> t) & 1 == 1, rolled, cur)
    tsk_refs[0][...] = cur
    for a in range(1, _A):
        tsk_refs[a][...] = jnp.concatenate(
            [cur[:, :, -a:], cur[:, :, :-a]], axis=2)

    lane2 = lax.broadcasted_iota(jnp.int32, (_B, _N), 1)

    t0row = tsk_refs[0][pl.ds(0, 1)][0]
    t00 = jnp.broadcast_to(t0row[:, 0:1], (_B, _N))
    p1 = tuple(
        jnp.where(lane2 == a, t00, f32(_BIG)) for a in range(_A + 1))
    p2 = tuple(jnp.full((_B, _N), _BIG, f32) for _ in range(_A))

    def fwd_body(k, carry):
        p1, p2, kmj = carry
        r = jnp.where(k < _N, k, k - _N)
        ts = [ref[pl.ds(r, 1)][0] for ref in tsk_refs]
        vmin0 = jnp.minimum(jnp.minimum(p1[1], p2[0]), p1[0])
        wa = jnp.exp2(vmin0 - p1[1])
        wb = jnp.exp2(vmin0 - p2[0])
        wc = jnp.exp2(vmin0 - p1[0])
        z = (wa + wb) + wc
        v0 = (ts[0] + vmin0) - jnp.log2(z)
        vs = [v0]
        for a in range(1, _A):
            vmin = jnp.minimum(jnp.minimum(p1[a + 1], p2[a]), p1[a])
            za = (jnp.exp2(vmin - p1[a + 1]) + jnp.exp2(vmin - p2[a])
                  + jnp.exp2(vmin - p1[a]))
            vs.append((ts[a] + vmin) - jnp.log2(za))
        v_top = jnp.where(lane2 < _A, f32(_BIG), _rot_r(v0, _A))
        vs.append(v_top)
        valid = (kmj >= 0) & (kmj <= _N - 1)
        rz = jnp.where(valid, 1.0 / z, f32(0.0))
        qa0 = wa * rz
        qb0 = wb * rz
        qc0 = wc * rz
        for a in range(_A):
            qas_refs[a][pl.ds(k, 1)] = jnp.where(
                lane2 < _N - (a + 1), _rot_l(qa0, a + 1), f32(0.0))[None]
            qbs_refs[a][pl.ds(k, 1)] = jnp.where(
                lane2 < _N - (a + 1), _rot_l(qb0, a + 1), f32(0.0))[None]
            if a == 0:
                qc_refs[0][pl.ds(k, 1)] = qc0[None]
            else:
                qc_refs[a][pl.ds(k, 1)] = jnp.where(
                    lane2 < _N - a, _rot_l(qc0, a), f32(0.0))[None]
        return (tuple(vs), tuple(p1[1 : _A + 1]), kmj + 1)

    kmj0 = 1 - lane2
    lax.fori_loop(1, _ND, fwd_body, (p1, p2, kmj0), unroll=4)

    zeros_row = jnp.zeros((1, _B, _N), f32)
    for a in range(_A):
        qbs_refs[a][pl.ds(_ND, 1)] = zeros_row

    e_seed = jnp.where(lane2 == _N - 1, f32(1.0), f32(0.0))
    esk_ref[pl.ds(_ND - 1, 1)] = e_seed[None]

    def bwd_body(s, carry):
        q1, q2 = carry
        k = _ND - 2 - s
        es = []
        for a in range(_A):
            qas = qas_refs[a][pl.ds(k + 1, 1)][0]
            qbs = qbs_refs[a][pl.ds(k + 2, 1)][0]
            qc = qc_refs[a][pl.ds(k + 1, 1)][0]
            es.append((qas * q1[a + 1] + qbs * q2[a]) + qc * q1[a])
        e_top = jnp.where(lane2 >= _N - _A, f32(0.0), _rot_l(es[0], _A))
        es.append(e_top)
        esk_ref[pl.ds(k, 1)] = es[0][None]
        return (tuple(es), tuple(q1[1 : _A + 1]))

    q1_init = tuple(
        jnp.where(lane2 == _N - 1 - a, f32(1.0), f32(0.0))
        for a in range(_A + 1))
    q2_init = tuple(jnp.zeros((_B, _N), f32) for _ in range(_A))
    lax.fori_loop(0, _ND - 1, bwd_body, (q1_init, q2_init), unroll=4)

    cur2 = jnp.mean(esk_ref[...], axis=1)
    lane2b = lax.broadcasted_iota(jnp.int32, (2 * _N, _N), 1)
    for t in range(7):
        s = 1 << t
        rolled = jnp.concatenate([cur2[s:], cur2[:s]], axis=0)
        cur2 = jnp.where((lane2b >> t) & 1 == 1, rolled, cur2)
    out_ref[...] = cur2[:_N]


@functools.partial(jax.jit, static_argnames=())
def _dtw_batch(D):
    Dt = jnp.transpose(D, (1, 0, 2))
    scratch = [pltpu.VMEM((_N, _B, _N), jnp.float32)] * _A
    scratch += [pltpu.VMEM((2 * _N, _B, _N), jnp.float32)] * (3 * _A + 1)
    return pl.pallas_call(
        _dtw_kernel,
        out_shape=jax.ShapeDtypeStruct((_N, _N), jnp.float32),
        in_specs=[pl.BlockSpec(memory_space=pltpu.VMEM)],
        out_specs=pl.BlockSpec(memory_space=pltpu.VMEM),
        scratch_shapes=scratch,
    )(Dt)


def kernel(D):
    return _dtw_batch(D)
